# baseline (speedup 1.0000x reference)
import jax
import jax.numpy as jnp
from jax import lax
from jax.experimental import pallas as pl
from jax.experimental.pallas import tpu as pltpu
from jax.experimental.pallas import tpu_sc as plsc

B, S, V = 64, 8, 100000
R = B * S
NC, NS, L = 2, 16, 16
NW = NC * NS
B_PER_W = B // NW
ROWS_PER_W = B_PER_W * S
TILE = 128

NT = 22
CW = NT * TILE
NCH = 24
COLS_SC = NCH * CW
KC = TILE // L

CB = 512
BB = 64
TC0B = COLS_SC // CB
COLS_TCEND = (V // CB) * CB
NB = (COLS_TCEND - COLS_SC) // CB

TAIL0 = COLS_TCEND
TAILW = 2 * TILE

_NEG_INF = float("-inf")


def _sc_body(x_hbm, tail_hbm, id_hbm, val_hbm,
             buf0, buf1, tail_buf, accv, acci, val_mat, idx_mat, oid, oval,
             sem0, sem1, sem_t):
    wid = lax.axis_index("s") * NC + lax.axis_index("c")
    row0 = wid * ROWS_PER_W
    bufs = (buf0, buf1)
    sems = (sem0, sem1)
    lane = lax.broadcasted_iota(jnp.int32, (L,), 0)

    def start(b, w, slot):
        for t in range(NT):
            pltpu.make_async_copy(
                x_hbm.at[b, :, pl.ds(w * CW + t * TILE, TILE)],
                bufs[slot].at[t], sems[slot]).start()

    def wait(slot):
        for t in range(NT):
            pltpu.make_async_copy(
                x_hbm.at[0, :, pl.ds(0, TILE)],
                bufs[slot].at[0], sems[slot]).wait()

    def process_chunk(buf, civ0):
        def s_body(s, carry):
            bs = [accv[s, c, :] for c in range(KC)]
            bis = [acci[s, c, :] for c in range(KC)]
            for tt in range(NT):
                civ_t = civ0 + tt
                for c in range(KC):
                    x = buf[tt, s, pl.ds(c * L, L)]
                    m = x > bs[c]
                    bs[c] = jnp.maximum(bs[c], x)
                    bis[c] = jnp.where(m, civ_t, bis[c])
            for c in range(KC):
                accv[s, c, :] = bs[c]
                acci[s, c, :] = bis[c]
            return carry

        lax.fori_loop(0, S, s_body, 0)
        return civ0 + NT

    for b_local in range(B_PER_W):
        b = wid * B_PER_W + b_local
        for s in range(S):
            for c in range(KC):
                accv[s, c, :] = jnp.full((L,), _NEG_INF, jnp.float32)
                acci[s, c, :] = jnp.zeros((L,), jnp.int32)
        start(b, 0, 0)
        start(b, 1, 1)
        tail_cp = pltpu.make_async_copy(tail_hbm.at[b], tail_buf, sem_t)
        tail_cp.start()

        def pair_body(j, civ0):
            for par in range(2):
                w = 2 * j + par
                wait(par)
                civ0 = process_chunk(bufs[par], civ0)

                @pl.when(w + 2 < NCH)
                def _prefetch():
                    start(b, w + 2, par)
            return civ0

        lax.fori_loop(0, NCH // 2, pair_body, jnp.zeros((L,), jnp.int32))

        tail_cp.wait()
        for s in range(S):
            best = accv[s, 0, :]
            bidx = acci[s, 0, :] * TILE + lane
            for c in range(1, KC):
                bv = accv[s, c, :]
                bi = acci[s, c, :] * TILE + (c * L) + lane
                better = (bv > best) | ((bv == best) & (bi < bidx))
                best = jnp.where(better, bv, best)
                bidx = jnp.where(better, bi, bidx)
            for t in range(TAILW // L):
                x = tail_buf[s, pl.ds(t * L, L)]
                ci = lane + (TAIL0 + t * L)
                m = x > best
                best = jnp.maximum(best, x)
                bidx = jnp.where(m, ci, bidx)
            r = b_local * S + s
            val_mat[r, :] = best
            idx_mat[r, :] = bidx

    best = plsc.load_gather(val_mat, [lane, jnp.zeros((L,), jnp.int32)])
    bidx = plsc.load_gather(idx_mat, [lane, jnp.zeros((L,), jnp.int32)])
    for j in range(1, L):
        col = jnp.full((L,), j, jnp.int32)
        bv = plsc.load_gather(val_mat, [lane, col])
        bi = plsc.load_gather(idx_mat, [lane, col])
        better = (bv > best) | ((bv == best) & (bi < bidx))
        best = jnp.where(better, bv, best)
        bidx = jnp.where(better, bi, bidx)
    oid[...] = bidx
    oval[...] = best
    pltpu.sync_copy(oid, id_hbm.at[pl.ds(row0, ROWS_PER_W)])
    pltpu.sync_copy(oval, val_hbm.at[pl.ds(row0, ROWS_PER_W)])


def _sc_call(logits, tail):
    mesh = plsc.VectorSubcoreMesh(
        core_axis_name="c", subcore_axis_name="s", num_cores=NC, num_subcores=NS)
    run = pl.kernel(
        _sc_body,
        out_type=(
            jax.ShapeDtypeStruct((R,), jnp.int32),
            jax.ShapeDtypeStruct((R,), jnp.float32),
        ),
        mesh=mesh,
        compiler_params=pltpu.CompilerParams(needs_layout_passes=False),
        scratch_types=(
            pltpu.VMEM((NT, S, TILE), jnp.float32),
            pltpu.VMEM((NT, S, TILE), jnp.float32),
            pltpu.VMEM((S, TAILW), jnp.float32),
            pltpu.VMEM((S, KC, L), jnp.float32),
            pltpu.VMEM((S, KC, L), jnp.int32),
            pltpu.VMEM((ROWS_PER_W, L), jnp.float32),
            pltpu.VMEM((ROWS_PER_W, L), jnp.int32),
            pltpu.VMEM((ROWS_PER_W,), jnp.int32),
            pltpu.VMEM((ROWS_PER_W,), jnp.float32),
            pltpu.SemaphoreType.DMA,
            pltpu.SemaphoreType.DMA,
            pltpu.SemaphoreType.DMA,
        ),
    )
    return run(logits, tail)


def _tc_body(x_ref, accv_ref, accc_ref):
    ci = pl.program_id(0)
    x = x_ref[...]

    def fold(av, ag, bv, bg):
        m = av >= bv
        return jnp.where(m, av, bv), jnp.where(m, ag, bg)

    g = [jnp.full((BB, S, TILE), c, jnp.int32) for c in range(4)]
    xs = [x[:, :, c * TILE:(c + 1) * TILE] for c in range(4)]
    v01, g01 = fold(xs[0], g[0], xs[1], g[1])
    v23, g23 = fold(xs[2], g[2], xs[3], g[3])
    rv, rg = fold(v01, g01, v23, g23)
    code = rg + 4 * (TC0B + ci)

    av = accv_ref[...]
    m = (rv > av) | (ci == 0)
    accv_ref[...] = jnp.where(m, rv, av)
    accc_ref[...] = jnp.where(m, code, accc_ref[...])


def _tc_resolve_body(accv_ref, accc_ref, mx_ref, idx_ref):
    av = accv_ref[...]
    ac = accc_ref[...]
    gmax = jnp.max(av, axis=-1)
    colv = ac * TILE + lax.broadcasted_iota(jnp.int32, (BB, S, TILE), 2)
    cand = jnp.where(av == gmax[..., None], colv, jnp.int32(V))
    mx_ref[...] = gmax
    idx_ref[...] = jnp.min(cand, axis=-1)


def _tc_call(logits):
    accv, accc = pl.pallas_call(
        _tc_body,
        grid=(NB,),
        in_specs=[pl.BlockSpec((BB, S, CB), lambda c: (0, 0, TC0B + c))],
        out_specs=[
            pl.BlockSpec((BB, S, TILE), lambda c: (0, 0, 0)),
            pl.BlockSpec((BB, S, TILE), lambda c: (0, 0, 0)),
        ],
        out_shape=[
            jax.ShapeDtypeStruct((B, S, TILE), jnp.float32),
            jax.ShapeDtypeStruct((B, S, TILE), jnp.int32),
        ],
        compiler_params=pltpu.CompilerParams(
            dimension_semantics=("arbitrary",)),
    )(logits)
    return pl.pallas_call(
        _tc_resolve_body,
        in_specs=[
            pl.BlockSpec((BB, S, TILE), lambda: (0, 0, 0)),
            pl.BlockSpec((BB, S, TILE), lambda: (0, 0, 0)),
        ],
        out_specs=[
            pl.BlockSpec((BB, S), lambda: (0, 0)),
            pl.BlockSpec((BB, S), lambda: (0, 0)),
        ],
        out_shape=[
            jax.ShapeDtypeStruct((B, S), jnp.float32),
            jax.ShapeDtypeStruct((B, S), jnp.int32),
        ],
    )(accv, accc)


@jax.jit
def kernel(logits):
    tail = jnp.pad(
        logits[:, :, TAIL0:], ((0, 0), (0, 0), (0, TAILW - (V - TAIL0))),
        constant_values=_NEG_INF)
    sc_id, sc_val = _sc_call(logits, tail)
    tc_val, tc_id = _tc_call(logits)
    sc_id = sc_id.reshape(B, S)
    sc_val = sc_val.reshape(B, S)
    better_sc = (sc_val > tc_val) | ((sc_val == tc_val) & (sc_id < tc_id))
    token_id = jnp.where(better_sc, sc_id, tc_id)
    token_logit = jnp.where(better_sc, sc_val, tc_val)
    return token_id, token_logit


# speedup vs baseline: 1.0145x; 1.0145x over previous
import jax
import jax.numpy as jnp
from jax import lax
from jax.experimental import pallas as pl
from jax.experimental.pallas import tpu as pltpu
from jax.experimental.pallas import tpu_sc as plsc

B, S, V = 64, 8, 100000
R = B * S
NC, NS, L = 2, 16, 16
NW = NC * NS
B_PER_W = B // NW
ROWS_PER_W = B_PER_W * S
TILE = 128

NT = 22
CW = NT * TILE
NCH = 22
COLS_SC = NCH * CW
KC = TILE // L

CB = 512
BB = 64
TC0B = COLS_SC // CB
COLS_TCEND = (V // CB) * CB
NB = (COLS_TCEND - COLS_SC) // CB

TAIL0 = (V // TILE) * TILE
TAILW = V - TAIL0
MID0 = COLS_TCEND
NTAILT = (V - COLS_TCEND) // TILE

_NEG_INF = float("-inf")


def _sc_body(x_hbm, id_hbm, val_hbm,
             buf0, buf1, tailm_buf, tailr_buf, accv, acci, val_mat, idx_mat,
             oid, oval, sem0, sem1, sem_t):
    wid = lax.axis_index("s") * NC + lax.axis_index("c")
    row0 = wid * ROWS_PER_W
    bufs = (buf0, buf1)
    sems = (sem0, sem1)
    lane = lax.broadcasted_iota(jnp.int32, (L,), 0)

    def start(b, w, slot):
        for t in range(NT):
            pltpu.make_async_copy(
                x_hbm.at[b, :, pl.ds(w * CW + t * TILE, TILE)],
                bufs[slot].at[t], sems[slot]).start()

    def wait(slot):
        for t in range(NT):
            pltpu.make_async_copy(
                x_hbm.at[0, :, pl.ds(0, TILE)],
                bufs[slot].at[0], sems[slot]).wait()

    def process_chunk(buf, civ0):
        def s_body(s, carry):
            bs = [accv[s, c, :] for c in range(KC)]
            bis = [acci[s, c, :] for c in range(KC)]
            for tt in range(NT):
                civ_t = civ0 + tt
                for c in range(KC):
                    x = buf[tt, s, pl.ds(c * L, L)]
                    m = x > bs[c]
                    bs[c] = jnp.maximum(bs[c], x)
                    bis[c] = jnp.where(m, civ_t, bis[c])
            for c in range(KC):
                accv[s, c, :] = bs[c]
                acci[s, c, :] = bis[c]
            return carry

        lax.fori_loop(0, S, s_body, 0)
        return civ0 + NT

    for b_local in range(B_PER_W):
        b = wid * B_PER_W + b_local
        for s in range(S):
            for c in range(KC):
                accv[s, c, :] = jnp.full((L,), _NEG_INF, jnp.float32)
                acci[s, c, :] = jnp.zeros((L,), jnp.int32)
        start(b, 0, 0)
        start(b, 1, 1)
        tail_cp1 = pltpu.make_async_copy(
            x_hbm.at[b, :, pl.ds(MID0, TILE)], tailm_buf, sem_t)
        tail_cp2 = pltpu.make_async_copy(
            x_hbm.at[b, :, pl.ds(TAIL0, TAILW)], tailr_buf, sem_t)
        tail_cp1.start()
        tail_cp2.start()

        def pair_body(j, civ0):
            for par in range(2):
                w = 2 * j + par
                wait(par)
                civ0 = process_chunk(bufs[par], civ0)

                @pl.when(w + 2 < NCH)
                def _prefetch():
                    start(b, w + 2, par)
            return civ0

        lax.fori_loop(0, NCH // 2, pair_body, jnp.zeros((L,), jnp.int32))

        tail_cp1.wait()
        tail_cp2.wait()
        for s in range(S):
            best = accv[s, 0, :]
            bidx = acci[s, 0, :] * TILE + lane
            for c in range(1, KC):
                bv = accv[s, c, :]
                bi = acci[s, c, :] * TILE + (c * L) + lane
                better = (bv > best) | ((bv == best) & (bi < bidx))
                best = jnp.where(better, bv, best)
                bidx = jnp.where(better, bi, bidx)
            for t in range(TILE // L):
                x = tailm_buf[s, pl.ds(t * L, L)]
                ci = lane + (MID0 + t * L)
                m = x > best
                best = jnp.maximum(best, x)
                bidx = jnp.where(m, ci, bidx)
            for t in range(TAILW // L):
                x = tailr_buf[s, pl.ds(t * L, L)]
                ci = lane + (TAIL0 + t * L)
                m = x > best
                best = jnp.maximum(best, x)
                bidx = jnp.where(m, ci, bidx)
            r = b_local * S + s
            val_mat[r, :] = best
            idx_mat[r, :] = bidx

    best = plsc.load_gather(val_mat, [lane, jnp.zeros((L,), jnp.int32)])
    bidx = plsc.load_gather(idx_mat, [lane, jnp.zeros((L,), jnp.int32)])
    for j in range(1, L):
        col = jnp.full((L,), j, jnp.int32)
        bv = plsc.load_gather(val_mat, [lane, col])
        bi = plsc.load_gather(idx_mat, [lane, col])
        better = (bv > best) | ((bv == best) & (bi < bidx))
        best = jnp.where(better, bv, best)
        bidx = jnp.where(better, bi, bidx)
    oid[...] = bidx
    oval[...] = best
    pltpu.sync_copy(oid, id_hbm.at[pl.ds(row0, ROWS_PER_W)])
    pltpu.sync_copy(oval, val_hbm.at[pl.ds(row0, ROWS_PER_W)])


def _sc_call(logits):
    mesh = plsc.VectorSubcoreMesh(
        core_axis_name="c", subcore_axis_name="s", num_cores=NC, num_subcores=NS)
    run = pl.kernel(
        _sc_body,
        out_type=(
            jax.ShapeDtypeStruct((R,), jnp.int32),
            jax.ShapeDtypeStruct((R,), jnp.float32),
        ),
        mesh=mesh,
        compiler_params=pltpu.CompilerParams(needs_layout_passes=False),
        scratch_types=(
            pltpu.VMEM((NT, S, TILE), jnp.float32),
            pltpu.VMEM((NT, S, TILE), jnp.float32),
            pltpu.VMEM((S, TILE), jnp.float32),
            pltpu.VMEM((S, TAILW), jnp.float32),
            pltpu.VMEM((S, KC, L), jnp.float32),
            pltpu.VMEM((S, KC, L), jnp.int32),
            pltpu.VMEM((ROWS_PER_W, L), jnp.float32),
            pltpu.VMEM((ROWS_PER_W, L), jnp.int32),
            pltpu.VMEM((ROWS_PER_W,), jnp.int32),
            pltpu.VMEM((ROWS_PER_W,), jnp.float32),
            pltpu.SemaphoreType.DMA,
            pltpu.SemaphoreType.DMA,
            pltpu.SemaphoreType.DMA,
        ),
    )
    return run(logits)


def _tc_body(x_ref, accv_ref, accc_ref):
    ci = pl.program_id(0)
    x = x_ref[...]

    def fold(av, ag, bv, bg):
        m = av >= bv
        return jnp.where(m, av, bv), jnp.where(m, ag, bg)

    g = [jnp.full((BB, S, TILE), c, jnp.int32) for c in range(4)]
    xs = [x[:, :, c * TILE:(c + 1) * TILE] for c in range(4)]
    v01, g01 = fold(xs[0], g[0], xs[1], g[1])
    v23, g23 = fold(xs[2], g[2], xs[3], g[3])
    rv, rg = fold(v01, g01, v23, g23)
    code = rg + 4 * (TC0B + ci)

    av = accv_ref[...]
    m = (rv > av) | (ci == 0)
    accv_ref[...] = jnp.where(m, rv, av)
    accc_ref[...] = jnp.where(m, code, accc_ref[...])


def _tc_resolve_body(accv_ref, accc_ref, scid_ref, scval_ref, id_ref, lg_ref):
    av = accv_ref[...]
    ac = accc_ref[...]
    gmax = jnp.max(av, axis=-1)
    colv = ac * TILE + lax.broadcasted_iota(jnp.int32, (BB, S, TILE), 2)
    cand = jnp.where(av == gmax[..., None], colv, jnp.int32(V))
    tc_idx = jnp.min(cand, axis=-1)
    sc_id = scid_ref[...]
    sc_val = scval_ref[...]
    better_sc = (sc_val > gmax) | ((sc_val == gmax) & (sc_id < tc_idx))
    id_ref[...] = jnp.where(better_sc, sc_id, tc_idx)
    lg_ref[...] = jnp.where(better_sc, sc_val, gmax)


def _tc_call(logits, sc_id, sc_val):
    accv, accc = pl.pallas_call(
        _tc_body,
        grid=(NB,),
        in_specs=[pl.BlockSpec((BB, S, CB), lambda c: (0, 0, TC0B + c))],
        out_specs=[
            pl.BlockSpec((BB, S, TILE), lambda c: (0, 0, 0)),
            pl.BlockSpec((BB, S, TILE), lambda c: (0, 0, 0)),
        ],
        out_shape=[
            jax.ShapeDtypeStruct((B, S, TILE), jnp.float32),
            jax.ShapeDtypeStruct((B, S, TILE), jnp.int32),
        ],
        compiler_params=pltpu.CompilerParams(
            dimension_semantics=("arbitrary",)),
    )(logits)
    return pl.pallas_call(
        _tc_resolve_body,
        in_specs=[
            pl.BlockSpec((BB, S, TILE), lambda: (0, 0, 0)),
            pl.BlockSpec((BB, S, TILE), lambda: (0, 0, 0)),
            pl.BlockSpec((B, S), lambda: (0, 0)),
            pl.BlockSpec((B, S), lambda: (0, 0)),
        ],
        out_specs=[
            pl.BlockSpec((B, S), lambda: (0, 0)),
            pl.BlockSpec((B, S), lambda: (0, 0)),
        ],
        out_shape=[
            jax.ShapeDtypeStruct((B, S), jnp.int32),
            jax.ShapeDtypeStruct((B, S), jnp.float32),
        ],
    )(accv, accc, sc_id, sc_val)


@jax.jit
def kernel(logits):
    sc_id, sc_val = _sc_call(logits)
    token_id, token_logit = _tc_call(
        logits, sc_id.reshape(B, S), sc_val.reshape(B, S))
    return token_id, token_logit


# speedup vs baseline: 1.0474x; 1.0325x over previous
import jax
import jax.numpy as jnp
from jax import lax
from jax.experimental import pallas as pl
from jax.experimental.pallas import tpu as pltpu
from jax.experimental.pallas import tpu_sc as plsc

B, S, V = 64, 8, 100000
R = B * S
NC, NS, L = 2, 16, 16
NW = NC * NS
B_PER_W = B // NW
ROWS_PER_W = B_PER_W * S
TILE = 128

NT = 22
CW = NT * TILE
NCH = 22
COLS_SC = NCH * CW
KC = TILE // L

CB = 512
BB = 64
TC0B = COLS_SC // CB
COLS_TCEND = (V // CB) * CB
NB = (COLS_TCEND - COLS_SC) // CB

TAIL0 = (V // TILE) * TILE
TAILW = V - TAIL0
MID0 = COLS_TCEND
NTAILT = (V - COLS_TCEND) // TILE

_NEG_INF = float("-inf")


def _sc_body(x_hbm, id_hbm, val_hbm,
             buf0, buf1, tailm_buf, tailr_buf, accv, acci, val_mat, idx_mat,
             oid, oval, sem0, sem1, sem_t):
    wid = lax.axis_index("s") * NC + lax.axis_index("c")
    row0 = wid * ROWS_PER_W
    bufs = (buf0, buf1)
    sems = (sem0, sem1)
    lane = lax.broadcasted_iota(jnp.int32, (L,), 0)

    def start(b, w, slot):
        for t in range(NT):
            pltpu.make_async_copy(
                x_hbm.at[b, :, pl.ds(w * CW + t * TILE, TILE)],
                bufs[slot].at[t], sems[slot]).start()

    def wait(slot):
        for t in range(NT):
            pltpu.make_async_copy(
                x_hbm.at[0, :, pl.ds(0, TILE)],
                bufs[slot].at[0], sems[slot]).wait()

    def process_chunk(buf, civ0):
        def s_body(s, carry):
            bs = [accv[s, c, :] for c in range(KC)]
            bis = [acci[s, c, :] for c in range(KC)]
            for tt in range(NT):
                civ_t = civ0 + tt
                for c in range(KC):
                    x = buf[tt, s, pl.ds(c * L, L)]
                    m = x > bs[c]
                    bs[c] = jnp.maximum(bs[c], x)
                    bis[c] = jnp.where(m, civ_t, bis[c])
            for c in range(KC):
                accv[s, c, :] = bs[c]
                acci[s, c, :] = bis[c]
            return carry

        lax.fori_loop(0, S, s_body, 0)
        return civ0 + NT

    for b_local in range(B_PER_W):
        b = wid * B_PER_W + b_local
        for s in range(S):
            for c in range(KC):
                accv[s, c, :] = jnp.full((L,), _NEG_INF, jnp.float32)
                acci[s, c, :] = jnp.zeros((L,), jnp.int32)
        start(b, 0, 0)
        start(b, 1, 1)
        tail_cp1 = pltpu.make_async_copy(
            x_hbm.at[b, :, pl.ds(MID0, TILE)], tailm_buf, sem_t)
        tail_cp2 = pltpu.make_async_copy(
            x_hbm.at[b, :, pl.ds(TAIL0, TAILW)], tailr_buf, sem_t)
        tail_cp1.start()
        tail_cp2.start()

        def pair_body(j, civ0):
            for par in range(2):
                w = 2 * j + par
                wait(par)
                civ0 = process_chunk(bufs[par], civ0)

                @pl.when(w + 2 < NCH)
                def _prefetch():
                    start(b, w + 2, par)
            return civ0

        lax.fori_loop(0, NCH // 2, pair_body, jnp.zeros((L,), jnp.int32))

        tail_cp1.wait()
        tail_cp2.wait()
        for s in range(S):
            best = accv[s, 0, :]
            bidx = acci[s, 0, :] * TILE + lane
            for c in range(1, KC):
                bv = accv[s, c, :]
                bi = acci[s, c, :] * TILE + (c * L) + lane
                better = (bv > best) | ((bv == best) & (bi < bidx))
                best = jnp.where(better, bv, best)
                bidx = jnp.where(better, bi, bidx)
            for t in range(TILE // L):
                x = tailm_buf[s, pl.ds(t * L, L)]
                ci = lane + (MID0 + t * L)
                m = x > best
                best = jnp.maximum(best, x)
                bidx = jnp.where(m, ci, bidx)
            for t in range(TAILW // L):
                x = tailr_buf[s, pl.ds(t * L, L)]
                ci = lane + (TAIL0 + t * L)
                m = x > best
                best = jnp.maximum(best, x)
                bidx = jnp.where(m, ci, bidx)
            r = b_local * S + s
            val_mat[r, :] = best
            idx_mat[r, :] = bidx

    best = plsc.load_gather(val_mat, [lane, jnp.zeros((L,), jnp.int32)])
    bidx = plsc.load_gather(idx_mat, [lane, jnp.zeros((L,), jnp.int32)])
    for j in range(1, L):
        col = jnp.full((L,), j, jnp.int32)
        bv = plsc.load_gather(val_mat, [lane, col])
        bi = plsc.load_gather(idx_mat, [lane, col])
        better = (bv > best) | ((bv == best) & (bi < bidx))
        best = jnp.where(better, bv, best)
        bidx = jnp.where(better, bi, bidx)
    oid[...] = bidx
    oval[...] = best
    pltpu.sync_copy(oid, id_hbm.at[pl.ds(row0, ROWS_PER_W)])
    pltpu.sync_copy(oval, val_hbm.at[pl.ds(row0, ROWS_PER_W)])


def _sc_call(logits):
    mesh = plsc.VectorSubcoreMesh(
        core_axis_name="c", subcore_axis_name="s", num_cores=NC, num_subcores=NS)
    run = pl.kernel(
        _sc_body,
        out_type=(
            jax.ShapeDtypeStruct((R,), jnp.int32),
            jax.ShapeDtypeStruct((R,), jnp.float32),
        ),
        mesh=mesh,
        compiler_params=pltpu.CompilerParams(needs_layout_passes=False),
        scratch_types=(
            pltpu.VMEM((NT, S, TILE), jnp.float32),
            pltpu.VMEM((NT, S, TILE), jnp.float32),
            pltpu.VMEM((S, TILE), jnp.float32),
            pltpu.VMEM((S, TAILW), jnp.float32),
            pltpu.VMEM((S, KC, L), jnp.float32),
            pltpu.VMEM((S, KC, L), jnp.int32),
            pltpu.VMEM((ROWS_PER_W, L), jnp.float32),
            pltpu.VMEM((ROWS_PER_W, L), jnp.int32),
            pltpu.VMEM((ROWS_PER_W,), jnp.int32),
            pltpu.VMEM((ROWS_PER_W,), jnp.float32),
            pltpu.SemaphoreType.DMA,
            pltpu.SemaphoreType.DMA,
            pltpu.SemaphoreType.DMA,
        ),
    )
    return run(logits)


def _tc_body(x_ref, accv_ref, accc_ref):
    ci = pl.program_id(0)
    x = x_ref[...]

    def fold(av, ag, bv, bg):
        m = av >= bv
        return jnp.where(m, av, bv), jnp.where(m, ag, bg)

    g = [jnp.full((BB, S, TILE), c, jnp.int32) for c in range(4)]
    xs = [x[:, :, c * TILE:(c + 1) * TILE] for c in range(4)]
    v01, g01 = fold(xs[0], g[0], xs[1], g[1])
    v23, g23 = fold(xs[2], g[2], xs[3], g[3])
    rv, rg = fold(v01, g01, v23, g23)
    code = rg + 4 * (TC0B + ci)

    av = accv_ref[...]
    m = (rv > av) | (ci == 0)
    accv_ref[...] = jnp.where(m, rv, av)
    accc_ref[...] = jnp.where(m, code, accc_ref[...])


def _tc_resolve_body(accv_ref, accc_ref, mx_ref, idx_ref):
    av = accv_ref[...]
    ac = accc_ref[...]
    gmax = jnp.max(av, axis=-1)
    colv = ac * TILE + lax.broadcasted_iota(jnp.int32, (BB, S, TILE), 2)
    cand = jnp.where(av == gmax[..., None], colv, jnp.int32(V))
    mx_ref[...] = gmax
    idx_ref[...] = jnp.min(cand, axis=-1)


def _tc_call(logits):
    accv, accc = pl.pallas_call(
        _tc_body,
        grid=(NB,),
        in_specs=[pl.BlockSpec((BB, S, CB), lambda c: (0, 0, TC0B + c))],
        out_specs=[
            pl.BlockSpec((BB, S, TILE), lambda c: (0, 0, 0)),
            pl.BlockSpec((BB, S, TILE), lambda c: (0, 0, 0)),
        ],
        out_shape=[
            jax.ShapeDtypeStruct((B, S, TILE), jnp.float32),
            jax.ShapeDtypeStruct((B, S, TILE), jnp.int32),
        ],
        compiler_params=pltpu.CompilerParams(
            dimension_semantics=("arbitrary",)),
    )(logits)
    return pl.pallas_call(
        _tc_resolve_body,
        in_specs=[
            pl.BlockSpec((BB, S, TILE), lambda: (0, 0, 0)),
            pl.BlockSpec((BB, S, TILE), lambda: (0, 0, 0)),
        ],
        out_specs=[
            pl.BlockSpec((B, S), lambda: (0, 0)),
            pl.BlockSpec((B, S), lambda: (0, 0)),
        ],
        out_shape=[
            jax.ShapeDtypeStruct((B, S), jnp.float32),
            jax.ShapeDtypeStruct((B, S), jnp.int32),
        ],
    )(accv, accc)


@jax.jit
def kernel(logits):
    sc_id, sc_val = _sc_call(logits)
    tc_val, tc_id = _tc_call(logits)
    sc_id = sc_id.reshape(B, S)
    sc_val = sc_val.reshape(B, S)
    better_sc = (sc_val > tc_val) | ((sc_val == tc_val) & (sc_id < tc_id))
    token_id = jnp.where(better_sc, sc_id, tc_id)
    token_logit = jnp.where(better_sc, sc_val, tc_val)
    return token_id, token_logit
